# baseline (speedup 1.0000x reference)
import functools
import math

import jax
import jax.numpy as jnp
import numpy as np
from jax import lax
from jax.experimental import pallas as pl
from jax.experimental.pallas import tpu as pltpu
from jax.experimental.pallas import tpu_sc as plsc

SMOOTHING = 0.1
CONFIDENCE = 1.0 - SMOOTHING

TC_ROWS = 1024
TC_COLS = 4096


def _tc_kernel(x_ref, m_ref, acc_ref, msum_ref):
    i = pl.program_id(0)
    j = pl.program_id(1)

    @pl.when((i == 0) & (j == 0))
    def _init():
        acc_ref[...] = jnp.zeros((1, 1), jnp.float32)
        msum_ref[...] = jnp.zeros((1, 1), jnp.float32)

    xb = x_ref[...]
    mb = m_ref[0, 0, :]
    rsum = jnp.sum(xb, axis=1)
    acc_ref[...] += jnp.sum(rsum * mb).reshape(1, 1)

    @pl.when(j == 0)
    def _msum():
        msum_ref[...] += jnp.sum(mb).reshape(1, 1)


def _make_sc_gather(N, V):
    mesh = plsc.VectorSubcoreMesh(core_axis_name="c", subcore_axis_name="s",
                                  num_cores=1)
    NC = 1
    L = 16
    NW = 16
    rows_per_w = N // NW
    n_chunks = rows_per_w // L

    @functools.partial(
        pl.kernel,
        mesh=mesh,
        out_type=jax.ShapeDtypeStruct((NW, L), jnp.float32),
        scratch_types=[
            pltpu.VMEM((rows_per_w,), jnp.int32),
            pltpu.VMEM((rows_per_w,), jnp.float32),
            pltpu.VMEM((rows_per_w * L,), jnp.float32),
            pltpu.VMEM((L,), jnp.float32),
            pltpu.SemaphoreType.DMA,
            pltpu.SemaphoreType.DMA,
        ],
    )
    def sc_gather(x_hbm, t_hbm, m_hbm, out_hbm, t_v, m_v, seg_v, stage_v,
                  sem, sem2):
        wid = lax.axis_index("s") * NC + lax.axis_index("c")
        row0 = wid * rows_per_w
        cp_t = pltpu.async_copy(t_hbm.at[pl.ds(row0, rows_per_w)], t_v,
                                sem2)
        cp_m = pltpu.async_copy(m_hbm.at[pl.ds(row0, rows_per_w)], m_v,
                                sem2)
        cp_t.wait()
        cp_m.wait()

        iota = lax.iota(jnp.int32, L)
        acc = jnp.zeros((L,), jnp.float32)

        t_i = []
        copies = []
        for c in range(n_chunks):
            tv = t_v[pl.ds(c * L, L)]
            t_i.extend(tv[i] for i in range(L))
        for r in range(rows_per_w):
            c0 = pl.multiple_of(t_i[r] & ~(L - 1), L)
            copies.append(pltpu.async_copy(
                x_hbm.at[row0 + r, pl.ds(c0, L)],
                seg_v.at[pl.ds(r * L, L)], sem))
        for cp in copies:
            cp.wait()
        for c in range(n_chunks):
            mv = m_v[pl.ds(c * L, L)]
            for i in range(L):
                r = c * L + i
                seg = seg_v[pl.ds(r * L, L)]
                hit = iota == (t_i[r] & (L - 1))
                acc = acc + jnp.where(hit, seg, 0.0) * mv[i]

        stage_v[...] = acc
        pltpu.sync_copy(stage_v, out_hbm.at[wid])

    return sc_gather


def kernel(input, target, mask):
    B, T, V = input.shape
    N = B * T
    x = input.reshape(N, V)
    t = target.reshape(N).astype(jnp.int32)
    m = mask.reshape(N).astype(jnp.float32)

    fill = float(np.float32(SMOOTHING / (V - 1)))
    conf = CONFIDENCE
    dconf = float(np.float32(conf - fill))
    c_const = (V - 1) * fill * math.log(fill) + conf * math.log(conf)

    n_i = N // TC_ROWS
    n_j = V // TC_COLS
    m3 = m.reshape(n_i, 1, TC_ROWS)
    acc, msum = pl.pallas_call(
        _tc_kernel,
        grid=(n_i, n_j),
        in_specs=[
            pl.BlockSpec((TC_ROWS, TC_COLS), lambda i, j: (i, j)),
            pl.BlockSpec((1, 1, TC_ROWS), lambda i, j: (i, 0, 0)),
        ],
        out_specs=[
            pl.BlockSpec((1, 1), lambda i, j: (0, 0)),
            pl.BlockSpec((1, 1), lambda i, j: (0, 0)),
        ],
        out_shape=[
            jax.ShapeDtypeStruct((1, 1), jnp.float32),
            jax.ShapeDtypeStruct((1, 1), jnp.float32),
        ],
    )(x, m3)

    sc_gather = _make_sc_gather(N, V)
    gpart = sc_gather(x, t, m)
    g = jnp.sum(gpart)

    return (jnp.float32(c_const)
            - (fill * acc[0, 0] + dconf * g) / msum[0, 0])


# speedup vs baseline: 1.0163x; 1.0163x over previous
import functools
import math

import jax
import jax.numpy as jnp
import numpy as np
from jax import lax
from jax.experimental import pallas as pl
from jax.experimental.pallas import tpu as pltpu
from jax.experimental.pallas import tpu_sc as plsc

SMOOTHING = 0.1
CONFIDENCE = 1.0 - SMOOTHING

TC_ROWS = 1024
TC_COLS = 4096


def _tc_kernel(x_ref, m_ref, acc_ref, msum_ref):
    i = pl.program_id(0)
    j = pl.program_id(1)

    @pl.when((i == 0) & (j == 0))
    def _init():
        acc_ref[...] = jnp.zeros((1, 1), jnp.float32)
        msum_ref[...] = jnp.zeros((1, 1), jnp.float32)

    xb = x_ref[...]
    mb = m_ref[0, 0, :]
    rsum = jnp.sum(xb, axis=1)
    acc_ref[...] += jnp.sum(rsum * mb).reshape(1, 1)

    @pl.when(j == 0)
    def _msum():
        msum_ref[...] += jnp.sum(mb).reshape(1, 1)


def _make_sc_gather(N, V):
    mesh = plsc.VectorSubcoreMesh(core_axis_name="c", subcore_axis_name="s",
                                  num_cores=1)
    NC = 1
    L = 16
    NW = 16
    rows_per_w = N // NW
    n_chunks = rows_per_w // L

    @functools.partial(
        pl.kernel,
        mesh=mesh,
        out_type=jax.ShapeDtypeStruct((NW, L), jnp.float32),
        scratch_types=[
            pltpu.VMEM((rows_per_w,), jnp.int32),
            pltpu.VMEM((rows_per_w,), jnp.float32),
            pltpu.VMEM((rows_per_w * L,), jnp.float32),
            pltpu.VMEM((L,), jnp.float32),
            pltpu.SemaphoreType.DMA,
        ],
    )
    def sc_gather(x_hbm, t_hbm, m_hbm, out_hbm, t_v, m_v, seg_v, stage_v,
                  sem):
        wid = lax.axis_index("s") * NC + lax.axis_index("c")
        row0 = wid * rows_per_w
        pltpu.sync_copy(t_hbm.at[pl.ds(row0, rows_per_w)], t_v)
        pltpu.sync_copy(m_hbm.at[pl.ds(row0, rows_per_w)], m_v)

        iota = lax.iota(jnp.int32, L)
        acc = jnp.zeros((L,), jnp.float32)

        t_i = []
        copies = []
        for c in range(n_chunks):
            tv = t_v[pl.ds(c * L, L)]
            t_i.extend(tv[i] for i in range(L))
        for r in range(rows_per_w):
            c0 = pl.multiple_of(t_i[r] & ~(L - 1), L)
            copies.append(pltpu.async_copy(
                x_hbm.at[row0 + r, pl.ds(c0, L)],
                seg_v.at[pl.ds(r * L, L)], sem))
        for cp in copies:
            cp.wait()
        for c in range(n_chunks):
            mv = m_v[pl.ds(c * L, L)]
            for i in range(L):
                r = c * L + i
                seg = seg_v[pl.ds(r * L, L)]
                hit = iota == (t_i[r] & (L - 1))
                acc = acc + jnp.where(hit, seg, 0.0) * mv[i]

        stage_v[...] = acc
        pltpu.sync_copy(stage_v, out_hbm.at[wid])

    return sc_gather


def kernel(input, target, mask):
    B, T, V = input.shape
    N = B * T
    x = input.reshape(N, V)
    t = target.reshape(N).astype(jnp.int32)
    m = mask.reshape(N).astype(jnp.float32)

    fill = float(np.float32(SMOOTHING / (V - 1)))
    conf = CONFIDENCE
    dconf = float(np.float32(conf - fill))
    c_const = (V - 1) * fill * math.log(fill) + conf * math.log(conf)

    n_i = N // TC_ROWS
    n_j = V // TC_COLS
    m3 = m.reshape(n_i, 1, TC_ROWS)
    acc, msum = pl.pallas_call(
        _tc_kernel,
        grid=(n_i, n_j),
        in_specs=[
            pl.BlockSpec((TC_ROWS, TC_COLS), lambda i, j: (i, j)),
            pl.BlockSpec((1, 1, TC_ROWS), lambda i, j: (i, 0, 0)),
        ],
        out_specs=[
            pl.BlockSpec((1, 1), lambda i, j: (0, 0)),
            pl.BlockSpec((1, 1), lambda i, j: (0, 0)),
        ],
        out_shape=[
            jax.ShapeDtypeStruct((1, 1), jnp.float32),
            jax.ShapeDtypeStruct((1, 1), jnp.float32),
        ],
    )(x, m3)

    sc_gather = _make_sc_gather(N, V)
    gpart = sc_gather(x, t, m)
    g = jnp.sum(gpart)

    return (jnp.float32(c_const)
            - (fill * acc[0, 0] + dconf * g) / msum[0, 0])
